# baseline (speedup 1.0000x reference)
import jax
import jax.numpy as jnp
from jax.experimental import pallas as pl
from jax.experimental.pallas import tpu as pltpu

B = 2048
N_UP = 16
N_EL = 32
D_NODE = 64
D_EDGE = 16
D_STREAM = 32
NS = 16
JD = NS * D_STREAM
JE = NS * D_EDGE

BL = 1024
IC = 4


def _body(e_up_ref, e_dn_ref, nodes_ref,
          W2u_ref, bwu_ref, WhTu_ref, bhu_ref,
          W2d_ref, bwd_ref, WhTd_ref, bhd_ref,
          out_up_ref, out_dn_ref,
          hxs_u_ref, hxs_d_ref):
    ic = pl.program_id(1)

    @pl.when(ic == 0)
    def _compute_hx():
        for hxs_ref, WhT_ref, bh_ref, lo in (
                (hxs_u_ref, WhTu_ref, bhu_ref, 0),
                (hxs_d_ref, WhTd_ref, bhd_ref, N_UP)):
            WhT = WhT_ref[...]
            bh = bh_ref[...]
            for j in range(NS):
                hxs_ref[j * D_STREAM:(j + 1) * D_STREAM] = (
                    jnp.dot(WhT, nodes_ref[lo + j],
                            preferred_element_type=jnp.float32) + bh)

    def one_type(e_ref, W2_ref, bw_ref, hxs_ref, out_ref):
        W2 = W2_ref[...]
        bw = bw_ref[...]
        hxs = hxs_ref[...]
        for i in range(IC):
            e_i = e_ref[i].reshape(JE, BL)
            we_i = (jnp.dot(W2, e_i, preferred_element_type=jnp.float32)
                    + bw)
            m = we_i * hxs
            s = m[:256] + m[256:]
            s = s[:128] + s[128:]
            s = s[:64] + s[64:]
            out_ref[i] = s[:32] + s[32:]

    one_type(e_up_ref, W2u_ref, bwu_ref, hxs_u_ref, out_up_ref)
    one_type(e_dn_ref, W2d_ref, bwd_ref, hxs_d_ref, out_dn_ref)


@jax.jit
def kernel(nodes, edges_up, edges_down,
           W_w_up, b_w_up, W_h_up, b_h_up,
           W_w_down, b_w_down, W_h_down, b_h_down):
    et_up = edges_up.transpose(1, 2, 3, 0)
    et_dn = edges_down.transpose(1, 2, 3, 0)
    nt = nodes.transpose(1, 2, 0)

    scale = 1.0 / NS
    eye = jnp.eye(NS, dtype=jnp.float32)

    def w2_block_diag(w):
        t = eye[:, None, :, None] * w.T[None, :, None, :]
        return t.reshape(JD, JE)

    W2u = w2_block_diag(W_w_up)
    W2d = w2_block_diag(W_w_down)
    WhTu = W_h_up.T * scale
    WhTd = W_h_down.T * scale
    bwu = jnp.tile(b_w_up, NS)[:, None]
    bwd = jnp.tile(b_w_down, NS)[:, None]
    bhu = (b_h_up * scale)[:, None]
    bhd = (b_h_down * scale)[:, None]

    grid = (B // BL, N_EL // IC)
    bspec_e = pl.BlockSpec((IC, NS, D_EDGE, BL), lambda b, ic: (ic, 0, 0, b))
    bspec_n = pl.BlockSpec((N_EL, D_NODE, BL), lambda b, ic: (0, 0, b))
    bspec_W2 = pl.BlockSpec((JD, JE), lambda b, ic: (0, 0))
    bspec_WhT = pl.BlockSpec((D_STREAM, D_NODE), lambda b, ic: (0, 0))
    bspec_bw = pl.BlockSpec((JD, 1), lambda b, ic: (0, 0))
    bspec_bh = pl.BlockSpec((D_STREAM, 1), lambda b, ic: (0, 0))
    bspec_out = pl.BlockSpec((IC, D_STREAM, BL), lambda b, ic: (ic, 0, b))

    out_shape = (
        jax.ShapeDtypeStruct((N_EL, D_STREAM, B), jnp.float32),
        jax.ShapeDtypeStruct((N_EL, D_STREAM, B), jnp.float32),
    )
    out_up_t, out_dn_t = pl.pallas_call(
        _body,
        grid=grid,
        in_specs=[bspec_e, bspec_e, bspec_n,
                  bspec_W2, bspec_bw, bspec_WhT, bspec_bh,
                  bspec_W2, bspec_bw, bspec_WhT, bspec_bh],
        out_specs=[bspec_out, bspec_out],
        out_shape=out_shape,
        scratch_shapes=[pltpu.VMEM((JD, BL), jnp.float32),
                        pltpu.VMEM((JD, BL), jnp.float32)],
        compiler_params=pltpu.CompilerParams(
            dimension_semantics=("arbitrary", "arbitrary"),
            vmem_limit_bytes=100 * 1024 * 1024,
        ),
    )(et_up, et_dn, nt,
      W2u, bwu, WhTu, bhu,
      W2d, bwd, WhTd, bhd)
    return (out_up_t.transpose(2, 0, 1), out_dn_t.transpose(2, 0, 1))


# speedup vs baseline: 1.0721x; 1.0721x over previous
import jax
import jax.numpy as jnp
from jax.experimental import pallas as pl
from jax.experimental.pallas import tpu as pltpu

B = 2048
N_UP = 16
N_EL = 32
D_NODE = 64
D_EDGE = 16
D_STREAM = 32
NS = 16
JD = NS * D_STREAM
JE = NS * D_EDGE

BL = 256


def _body(e_up_ref, e_dn_ref, nodes_ref,
          W2u_ref, bwu_ref, WhTu_ref, bhu_ref,
          W2d_ref, bwd_ref, WhTd_ref, bhd_ref,
          out_up_ref, out_dn_ref):

    def one_type(e_ref, W2_ref, bw_ref, WhT_ref, bh_ref, sender_lo, out_ref):
        WhT = WhT_ref[...]
        bh = bh_ref[...]
        W2 = W2_ref[...]
        bw = bw_ref[...]
        hx_js = [
            jnp.dot(WhT, nodes_ref[sender_lo + j], preferred_element_type=jnp.float32) + bh
            for j in range(NS)
        ]
        hxs = jnp.concatenate(hx_js, axis=0)
        for i in range(N_EL):
            e_i = e_ref[i].reshape(JE, BL)
            we_i = (jnp.dot(W2, e_i, preferred_element_type=jnp.float32)
                    + bw)
            m = we_i * hxs
            s = m[:256] + m[256:]
            s = s[:128] + s[128:]
            s = s[:64] + s[64:]
            out_ref[i] = s[:32] + s[32:]

    one_type(e_up_ref, W2u_ref, bwu_ref, WhTu_ref, bhu_ref, 0, out_up_ref)
    one_type(e_dn_ref, W2d_ref, bwd_ref, WhTd_ref, bhd_ref, N_UP, out_dn_ref)


@jax.jit
def kernel(nodes, edges_up, edges_down,
           W_w_up, b_w_up, W_h_up, b_h_up,
           W_w_down, b_w_down, W_h_down, b_h_down):
    et_up = edges_up.transpose(1, 2, 3, 0)
    et_dn = edges_down.transpose(1, 2, 3, 0)
    nt = nodes.transpose(1, 2, 0)

    scale = 1.0 / NS
    eye = jnp.eye(NS, dtype=jnp.float32)

    def w2_block_diag(w):
        t = eye[:, None, :, None] * w.T[None, :, None, :]
        return t.reshape(JD, JE)

    W2u = w2_block_diag(W_w_up)
    W2d = w2_block_diag(W_w_down)
    WhTu = W_h_up.T * scale
    WhTd = W_h_down.T * scale
    bwu = jnp.tile(b_w_up, NS)[:, None]
    bwd = jnp.tile(b_w_down, NS)[:, None]
    bhu = (b_h_up * scale)[:, None]
    bhd = (b_h_down * scale)[:, None]

    grid = (B // BL,)
    bspec_e = pl.BlockSpec((N_EL, NS, D_EDGE, BL), lambda i: (0, 0, 0, i))
    bspec_n = pl.BlockSpec((N_EL, D_NODE, BL), lambda i: (0, 0, i))
    bspec_W2 = pl.BlockSpec((JD, JE), lambda i: (0, 0))
    bspec_WhT = pl.BlockSpec((D_STREAM, D_NODE), lambda i: (0, 0))
    bspec_bw = pl.BlockSpec((JD, 1), lambda i: (0, 0))
    bspec_bh = pl.BlockSpec((D_STREAM, 1), lambda i: (0, 0))
    bspec_out = pl.BlockSpec((N_EL, D_STREAM, BL), lambda i: (0, 0, i))

    out_shape = (
        jax.ShapeDtypeStruct((N_EL, D_STREAM, B), jnp.float32),
        jax.ShapeDtypeStruct((N_EL, D_STREAM, B), jnp.float32),
    )
    out_up_t, out_dn_t = pl.pallas_call(
        _body,
        grid=grid,
        in_specs=[bspec_e, bspec_e, bspec_n,
                  bspec_W2, bspec_bw, bspec_WhT, bspec_bh,
                  bspec_W2, bspec_bw, bspec_WhT, bspec_bh],
        out_specs=[bspec_out, bspec_out],
        out_shape=out_shape,
        compiler_params=pltpu.CompilerParams(
            dimension_semantics=("arbitrary",),
            vmem_limit_bytes=100 * 1024 * 1024,
        ),
    )(et_up, et_dn, nt,
      W2u, bwu, WhTu, bhu,
      W2d, bwd, WhTd, bhd)
    return (out_up_t.transpose(2, 0, 1), out_dn_t.transpose(2, 0, 1))


# speedup vs baseline: 1.2474x; 1.1635x over previous
import jax
import jax.numpy as jnp
from jax.experimental import pallas as pl
from jax.experimental.pallas import tpu as pltpu

B = 2048
N_UP = 16
N_EL = 32
D_NODE = 64
D_EDGE = 16
D_STREAM = 32
NS = 16
JD = NS * D_STREAM
JE = NS * D_EDGE

BL = 256


def _body(e_up_ref, e_dn_ref, nodes_ref,
          Wwu_ref, bwu_ref, Whu_ref, bhu_ref,
          Wwd_ref, bwd_ref, Whd_ref, bhd_ref,
          out_up_ref, out_dn_ref,
          W2u_ref, W2d_ref):
    step = pl.program_id(0)
    scale = 1.0 / NS

    @pl.when(step == 0)
    def _build_w2():
        for W2_ref, Ww_ref in ((W2u_ref, Wwu_ref), (W2d_ref, Wwd_ref)):
            wT = Ww_ref[...].T
            z16 = jnp.zeros((D_STREAM, D_EDGE), dtype=jnp.float32)
            rows = [jnp.concatenate([z16] * j + [wT] + [z16] * (NS - 1 - j),
                                    axis=1)
                    for j in range(NS)]
            W2_ref[...] = jnp.concatenate(rows, axis=0)

    def one_type(e_ref, W2_ref, bw_ref, Wh_ref, bh_ref, sender_lo, out_ref):
        WhT = Wh_ref[...].T * scale
        bh = bh_ref[...].T * scale
        W2 = W2_ref[...]
        bw_c = bw_ref[...].T
        bw = jnp.concatenate([bw_c] * NS, axis=0)
        hx_js = [
            jnp.dot(WhT, nodes_ref[sender_lo + j], preferred_element_type=jnp.float32) + bh
            for j in range(NS)
        ]
        hxs = jnp.concatenate(hx_js, axis=0)
        for i in range(N_EL):
            e_i = e_ref[i].reshape(JE, BL)
            we_i = (jnp.dot(W2, e_i, preferred_element_type=jnp.float32)
                    + bw)
            m = we_i * hxs
            s = m[:256] + m[256:]
            s = s[:128] + s[128:]
            s = s[:64] + s[64:]
            out_ref[i] = s[:32] + s[32:]

    one_type(e_up_ref, W2u_ref, bwu_ref, Whu_ref, bhu_ref, 0, out_up_ref)
    one_type(e_dn_ref, W2d_ref, bwd_ref, Whd_ref, bhd_ref, N_UP, out_dn_ref)


@jax.jit
def kernel(nodes, edges_up, edges_down,
           W_w_up, b_w_up, W_h_up, b_h_up,
           W_w_down, b_w_down, W_h_down, b_h_down):
    et_up = edges_up.transpose(1, 2, 3, 0)
    et_dn = edges_down.transpose(1, 2, 3, 0)
    nt = nodes.transpose(1, 2, 0)

    grid = (B // BL,)
    bspec_e = pl.BlockSpec((N_EL, NS, D_EDGE, BL), lambda i: (0, 0, 0, i))
    bspec_n = pl.BlockSpec((N_EL, D_NODE, BL), lambda i: (0, 0, i))
    bspec_Ww = pl.BlockSpec((D_EDGE, D_STREAM), lambda i: (0, 0))
    bspec_Wh = pl.BlockSpec((D_NODE, D_STREAM), lambda i: (0, 0))
    bspec_b = pl.BlockSpec((1, D_STREAM), lambda i: (0, 0))
    bspec_out = pl.BlockSpec((N_EL, D_STREAM, BL), lambda i: (0, 0, i))

    out_shape = (
        jax.ShapeDtypeStruct((N_EL, D_STREAM, B), jnp.float32),
        jax.ShapeDtypeStruct((N_EL, D_STREAM, B), jnp.float32),
    )
    out_up_t, out_dn_t = pl.pallas_call(
        _body,
        grid=grid,
        in_specs=[bspec_e, bspec_e, bspec_n,
                  bspec_Ww, bspec_b, bspec_Wh, bspec_b,
                  bspec_Ww, bspec_b, bspec_Wh, bspec_b],
        out_specs=[bspec_out, bspec_out],
        out_shape=out_shape,
        scratch_shapes=[pltpu.VMEM((JD, JE), jnp.float32),
                        pltpu.VMEM((JD, JE), jnp.float32)],
        compiler_params=pltpu.CompilerParams(
            dimension_semantics=("arbitrary",),
            vmem_limit_bytes=100 * 1024 * 1024,
        ),
    )(et_up, et_dn, nt,
      W_w_up, b_w_up[None], W_h_up, b_h_up[None],
      W_w_down, b_w_down[None], W_h_down, b_h_down[None])
    return (out_up_t.transpose(2, 0, 1), out_dn_t.transpose(2, 0, 1))
